# Initial kernel scaffold; baseline (speedup 1.0000x reference)
import jax
import jax.numpy as jnp
from jax.experimental import pallas as pl


def kernel(coords, torsions):
    raise NotImplementedError("write your pallas kernel here")


# speedup vs baseline: 10.0092x; 10.0092x over previous
import jax
import jax.numpy as jnp
from jax import lax
from jax.experimental import pallas as pl
from jax.experimental.pallas import tpu as pltpu
from jax.experimental.pallas import tpu_sc as plsc

_NC = 2
_NS = 16
_NW = _NC * _NS

_T = 1600
_CH = 128
_NCH = 4 * _T // _CH
_STEPS = _T // 16

_PI = 3.141592653589793
_ACOS = (1.5707963050, -0.2145988016, 0.0889789874, -0.0501743046,
         0.0308918810, -0.0170881256, 0.0066700901, -0.0012624911)


def _rsqrt(y):
    i = plsc.bitcast(y, jnp.int32)
    i = 0x5F3759DF - (i >> 1)
    r = plsc.bitcast(i, jnp.float32)
    for _ in range(3):
        r = r * (1.5 - 0.5 * y * r * r)
    return r


def _acos(x):
    ax = jnp.abs(x)
    u = 1.0 - ax
    su = u * _rsqrt(jnp.maximum(u, 1e-30))
    p = jnp.full((16,), _ACOS[7], jnp.float32)
    for c in _ACOS[6::-1]:
        p = p * ax + c
    r = su * p
    return jnp.where(x < 0.0, _PI - r, r)


def _torsion_body(coords_hbm, tors_hbm, out_hbm, idx_v, rows_v, phi_v, sem):
    wid = lax.axis_index("s") * _NC + lax.axis_index("c")
    nblk_total = tors_hbm.shape[0]
    base_n = nblk_total // _NW
    extra = nblk_total - base_n * _NW
    nblk_w = jnp.where(wid < extra, base_n + 1, base_n)

    lane4 = 4 * lax.broadcasted_iota(jnp.int32, (16,), 0)
    colc = [jnp.full((16,), c, jnp.int32) for c in range(3)]

    def block_body(j, carry):
        blk = wid + j * _NW
        pltpu.sync_copy(tors_hbm.at[blk], idx_v)
        pltpu.async_copy(coords_hbm.at[idx_v], rows_v, sem).wait()

        def step(s, carry2):
            rbase = 64 * s + lane4
            atoms = []
            for a in range(4):
                ra = rbase + a
                atoms.append([plsc.load_gather(rows_v, [ra, colc[c]])
                              for c in range(3)])
            (xi, yi, zi), (xj, yj, zj), (xk, yk, zk), (xl, yl, zl) = atoms
            b1x, b1y, b1z = xj - xi, yj - yi, zj - zi
            b2x, b2y, b2z = xk - xj, yk - yj, zk - zj
            b3x, b3y, b3z = xl - xk, yl - yk, zl - zk
            n1x = b1y * b2z - b1z * b2y
            n1y = b1z * b2x - b1x * b2z
            n1z = b1x * b2y - b1y * b2x
            n2x = b2y * b3z - b2z * b3y
            n2y = b2z * b3x - b2x * b3z
            n2z = b2x * b3y - b2y * b3x
            dot = n1x * n2x + n1y * n2y + n1z * n2z
            m1 = n1x * n1x + n1y * n1y + n1z * n1z
            m2 = n2x * n2x + n2y * n2y + n2z * n2z
            y = m1 * m2
            cos = jnp.clip(dot * _rsqrt(y), -1.0, 1.0)
            cos = jnp.where(y > 0.0, cos, jnp.float32(jnp.nan))
            phi = _acos(cos)
            d2 = n1x * b3x + n1y * b3y + n1z * b3z
            phi_v[pl.ds(s * 16, 16)] = jnp.where(d2 > 0.0, phi, -phi)
            return carry2

        lax.fori_loop(0, _STEPS, step, 0)
        pltpu.sync_copy(phi_v, out_hbm.at[pl.ds(blk * _T, _T)])
        return carry

    lax.fori_loop(0, nblk_w, block_body, 0)


def kernel(coords, torsions):
    n_tors = torsions.shape[0]
    nblk = n_tors // _T
    coords16 = jnp.pad(coords, ((0, 0), (0, 13)))
    tors3 = torsions.reshape(nblk, 4 * _T)

    launch = pl.kernel(
        _torsion_body,
        out_type=jax.ShapeDtypeStruct((n_tors,), jnp.float32),
        mesh=plsc.VectorSubcoreMesh(core_axis_name="c", subcore_axis_name="s"),
        scratch_types=[
            pltpu.VMEM((4 * _T,), jnp.int32),
            pltpu.VMEM((4 * _T, 16), jnp.float32),
            pltpu.VMEM((_T,), jnp.float32),
            pltpu.SemaphoreType.DMA,
        ],
        compiler_params=pltpu.CompilerParams(needs_layout_passes=False,
                                             use_tc_tiling_on_sc=False),
    )
    return launch(coords16, tors3)


# speedup vs baseline: 13.3390x; 1.3327x over previous
import jax
import jax.numpy as jnp
from jax import lax
from jax.experimental import pallas as pl
from jax.experimental.pallas import tpu as pltpu
from jax.experimental.pallas import tpu_sc as plsc

_NC = 2
_NS = 16
_NW = _NC * _NS

_T = 1600
_STEPS = _T // 16

_PI = 3.141592653589793
_ACOS = (1.5707963050, -0.2145988016, 0.0889789874, -0.0501743046,
         0.0308918810, -0.0170881256, 0.0066700901, -0.0012624911)


def _rsqrt(y):
    i = plsc.bitcast(y, jnp.int32)
    i = 0x5F3759DF - (i >> 1)
    r = plsc.bitcast(i, jnp.float32)
    for _ in range(3):
        r = r * (1.5 - 0.5 * y * r * r)
    return r


def _acos(x):
    ax = jnp.abs(x)
    u = 1.0 - ax
    su = u * _rsqrt(jnp.maximum(u, 1e-30))
    p = jnp.full((16,), _ACOS[7], jnp.float32)
    for c in _ACOS[6::-1]:
        p = p * ax + c
    r = su * p
    return jnp.where(x < 0.0, _PI - r, r)


def _torsion_body(xs_hbm, ys_hbm, zs_hbm, tors_hbm, out_hbm,
                  idx_v, xv, yv, zv, phi_v, sem):
    wid = lax.axis_index("s") * _NC + lax.axis_index("c")
    nblk_total = tors_hbm.shape[0] // (4 * _T)
    base_n = nblk_total // _NW
    extra = nblk_total - base_n * _NW
    nblk_w = jnp.where(wid < extra, base_n + 1, base_n)

    lane4 = 4 * lax.broadcasted_iota(jnp.int32, (16,), 0)

    def block_body(j, carry):
        blk = wid + j * _NW
        pltpu.sync_copy(tors_hbm.at[pl.ds(blk * 4 * _T, 4 * _T)], idx_v)
        cx = pltpu.async_copy(xs_hbm.at[idx_v], xv, sem)
        cy = pltpu.async_copy(ys_hbm.at[idx_v], yv, sem)
        cz = pltpu.async_copy(zs_hbm.at[idx_v], zv, sem)
        cx.wait()
        cy.wait()
        cz.wait()

        def step(s, carry2):
            rbase = 64 * s + lane4
            atoms = []
            for a in range(4):
                ra = rbase + a
                atoms.append([plsc.load_gather(v, [ra])
                              for v in (xv, yv, zv)])
            (xi, yi, zi), (xj, yj, zj), (xk, yk, zk), (xl, yl, zl) = atoms
            b1x, b1y, b1z = xj - xi, yj - yi, zj - zi
            b2x, b2y, b2z = xk - xj, yk - yj, zk - zj
            b3x, b3y, b3z = xl - xk, yl - yk, zl - zk
            n1x = b1y * b2z - b1z * b2y
            n1y = b1z * b2x - b1x * b2z
            n1z = b1x * b2y - b1y * b2x
            n2x = b2y * b3z - b2z * b3y
            n2y = b2z * b3x - b2x * b3z
            n2z = b2x * b3y - b2y * b3x
            dot = n1x * n2x + n1y * n2y + n1z * n2z
            m1 = n1x * n1x + n1y * n1y + n1z * n1z
            m2 = n2x * n2x + n2y * n2y + n2z * n2z
            y = m1 * m2
            cos = jnp.clip(dot * _rsqrt(y), -1.0, 1.0)
            cos = jnp.where(y > 0.0, cos, jnp.float32(jnp.nan))
            phi = _acos(cos)
            d2 = n1x * b3x + n1y * b3y + n1z * b3z
            phi_v[pl.ds(s * 16, 16)] = jnp.where(d2 > 0.0, phi, -phi)
            return carry2

        lax.fori_loop(0, _STEPS, step, 0)
        pltpu.sync_copy(phi_v, out_hbm.at[pl.ds(blk * _T, _T)])
        return carry

    lax.fori_loop(0, nblk_w, block_body, 0)


def kernel(coords, torsions):
    n_tors = torsions.shape[0]
    xs = coords[:, 0]
    ys = coords[:, 1]
    zs = coords[:, 2]
    tors_flat = torsions.reshape(-1)

    launch = pl.kernel(
        _torsion_body,
        out_type=jax.ShapeDtypeStruct((n_tors,), jnp.float32),
        mesh=plsc.VectorSubcoreMesh(core_axis_name="c", subcore_axis_name="s"),
        scratch_types=[
            pltpu.VMEM((4 * _T,), jnp.int32),
            pltpu.VMEM((4 * _T,), jnp.float32),
            pltpu.VMEM((4 * _T,), jnp.float32),
            pltpu.VMEM((4 * _T,), jnp.float32),
            pltpu.VMEM((_T,), jnp.float32),
            pltpu.SemaphoreType.DMA,
        ],
        compiler_params=pltpu.CompilerParams(needs_layout_passes=False,
                                             use_tc_tiling_on_sc=False),
    )
    return launch(xs, ys, zs, tors_flat)
